# baseline (speedup 1.0000x reference)
import functools

import jax
import jax.numpy as jnp
from jax import lax
from jax.experimental import pallas as pl
from jax.experimental.pallas import tpu as pltpu
from jax.experimental.pallas import tpu_sc as plsc

RADIUS = 0.07
K = 12
NUM_FREQ = 32
MAX_POS = 10000.0
NPAD = 10240
BP = 1024
BM = 64
BQ = 256
NW = 32
EPW = NPAD * K // NW
CHUNK = 128


def _embed(c):
    kf = lax.broadcasted_iota(jnp.int32, (1, NUM_FREQ), 1).astype(jnp.float32)
    freqs = 1.0 / (MAX_POS ** (kf / NUM_FREQ))
    parts = []
    for d in range(3):
        ang = c[:, d:d + 1] * freqs
        parts.append(jnp.sin(ang))
        parts.append(jnp.cos(ang))
    return jnp.concatenate(parts, axis=1)


def _prep_body(c_ref, w_ref, b_ref, o_ref):
    emb = _embed(c_ref[...])
    o_ref[...] = (
        jnp.dot(emb, w_ref[...], preferred_element_type=jnp.float32) + b_ref[...]
    )


def _prep(coords, w, b):
    return pl.pallas_call(
        _prep_body,
        grid=(NPAD // BP,),
        in_specs=[
            pl.BlockSpec((BP, 8), lambda i: (i, 0)),
            pl.BlockSpec((192, 128), lambda i: (0, 0)),
            pl.BlockSpec((1, 128), lambda i: (0, 0)),
        ],
        out_specs=pl.BlockSpec((BP, 128), lambda i: (i, 0)),
        out_shape=jax.ShapeDtypeStruct((NPAD, 128), jnp.float32),
    )(coords, w, b)


RWIN = (RADIUS * RADIUS + 6.0 / 512.0) ** 0.5
W = 3456
IMAX = 2147483647


def _search_body(x_ref, yw_ref, idx_ref, mask_ref):
    xb = x_ref[...]
    yw = yw_ref[...]
    xsq = (xb[:, 0:1] * xb[:, 0:1] + xb[:, 1:2] * xb[:, 1:2]) + xb[:, 2:3] * xb[:, 2:3]
    ysq = (yw[0:1, :] * yw[0:1, :] + yw[1:2, :] * yw[1:2, :]) + yw[2:3, :] * yw[2:3, :]
    oc = yw[3:4, :].astype(jnp.int32)
    prod = jnp.dot(xb, yw, preferred_element_type=jnp.float32)
    d2 = (xsq + ysq) - 2.0 * prod
    d2 = jnp.maximum(d2, 0.0)
    neg = jnp.where(d2 <= RADIUS * RADIUS, -d2, -jnp.inf)
    idx_ref[...] = jnp.zeros((BM, 128), jnp.int32)
    mask_ref[...] = jnp.zeros((BM, 128), jnp.float32)
    for kk in range(K):
        cm = jnp.max(neg, axis=1, keepdims=True)
        tie = neg == cm
        cidx = jnp.min(jnp.where(tie, oc, NPAD * 2), axis=1, keepdims=True)
        valid = cm > -jnp.inf
        idx_ref[:, kk:kk + 1] = jnp.where(valid, cidx, 0)
        mask_ref[:, kk:kk + 1] = jnp.where(valid, 1.0, 0.0)
        neg = jnp.where(tie & (oc == cidx), -jnp.inf, neg)


def _search(xp, ywin):
    return pl.pallas_call(
        _search_body,
        grid=(NPAD // BM,),
        in_specs=[
            pl.BlockSpec((BM, 8), lambda i: (i, 0)),
            pl.BlockSpec((8, W), lambda i: (i, 0)),
        ],
        out_specs=[
            pl.BlockSpec((BM, 128), lambda i: (i, 0)),
            pl.BlockSpec((BM, 128), lambda i: (i, 0)),
        ],
        out_shape=[
            jax.ShapeDtypeStruct((NPAD, 128), jnp.int32),
            jax.ShapeDtypeStruct((NPAD, 128), jnp.float32),
        ],
    )(xp, ywin)


NCH = EPW // CHUNK
NSLOT = 3


def _make_gather():
    mesh = plsc.VectorSubcoreMesh(core_axis_name="c", subcore_axis_name="s")

    @functools.partial(
        pl.kernel,
        mesh=mesh,
        out_type=(
            jax.ShapeDtypeStruct((NPAD * K, 128), jnp.float32),
            jax.ShapeDtypeStruct((NPAD * K, 128), jnp.float32),
        ),
        scratch_types=[
            pltpu.VMEM((EPW,), jnp.int32),
            pltpu.VMEM((NSLOT, CHUNK, 128), jnp.float32),
            pltpu.VMEM((NSLOT, CHUNK, 128), jnp.float32),
        ]
        + [pltpu.SemaphoreType.DMA] * (4 * NSLOT),
    )
    def gather(a1_hbm, fy_hbm, idx_hbm, g1_hbm, gf_hbm, idx_v, bufa, buff, *sems):
        ga = sems[0:NSLOT]
        gf = sems[NSLOT:2 * NSLOT]
        wsa = sems[2 * NSLOT:3 * NSLOT]
        wsf = sems[3 * NSLOT:4 * NSLOT]
        wid = lax.axis_index("s") * 2 + lax.axis_index("c")
        base = wid * EPW
        pltpu.sync_copy(idx_hbm.at[pl.ds(base, EPW)], idx_v)

        hA = [None] * NSLOT
        hF = [None] * NSLOT
        wA = [None] * NSLOT
        wF = [None] * NSLOT

        def g_start(c, s):
            isl = idx_v.at[pl.ds(c * CHUNK, CHUNK)]
            hA[s] = pltpu.async_copy(a1_hbm.at[isl], bufa.at[s], ga[s])
            hF[s] = pltpu.async_copy(fy_hbm.at[isl], buff.at[s], gf[s])

        for c in range(min(NSLOT, NCH)):
            g_start(c, c)
        for c in range(NCH):
            s = c % NSLOT
            hA[s].wait()
            hF[s].wait()
            wA[s] = pltpu.async_copy(
                bufa.at[s], g1_hbm.at[pl.ds(base + c * CHUNK, CHUNK)], wsa[s])
            wF[s] = pltpu.async_copy(
                buff.at[s], gf_hbm.at[pl.ds(base + c * CHUNK, CHUNK)], wsf[s])
            nc = c + NSLOT
            if nc < NCH:
                wA[s].wait()
                wF[s].wait()
                g_start(nc, s)
        for c in range(max(NCH - NSLOT, 0), NCH):
            s = c % NSLOT
            wA[s].wait()
            wF[s].wait()

    return gather


def _mlp_body(g1_ref, gf_ref, a2_ref, mk_ref, w2_ref, b2_ref, w3_ref, b3_ref, o_ref):
    a2 = a2_ref[...]
    w2 = w2_ref[...]
    b2 = b2_ref[...]
    w3 = w3_ref[...]
    b3 = b3_ref[...]
    mk = mk_ref[...]
    acc = jnp.zeros((BQ, 128), jnp.float32)
    for kk in range(K):
        h = jax.nn.gelu(g1_ref[kk] + a2)
        h = jax.nn.gelu(jnp.dot(h, w2, preferred_element_type=jnp.float32) + b2)
        kv = (jnp.dot(h, w3, preferred_element_type=jnp.float32) + b3) * gf_ref[kk]
        acc = acc + kv * mk[:, kk:kk + 1]
    o_ref[...] = acc


def _mlp(g1, gf, a2, mk, w2, b2, w3, b3):
    return pl.pallas_call(
        _mlp_body,
        grid=(NPAD // BQ,),
        in_specs=[
            pl.BlockSpec((K, BQ, 128), lambda i: (0, i, 0)),
            pl.BlockSpec((K, BQ, 128), lambda i: (0, i, 0)),
            pl.BlockSpec((BQ, 128), lambda i: (i, 0)),
            pl.BlockSpec((BQ, 128), lambda i: (i, 0)),
            pl.BlockSpec((128, 256), lambda i: (0, 0)),
            pl.BlockSpec((1, 256), lambda i: (0, 0)),
            pl.BlockSpec((256, 128), lambda i: (0, 0)),
            pl.BlockSpec((1, 128), lambda i: (0, 0)),
        ],
        out_specs=pl.BlockSpec((BQ, 128), lambda i: (i, 0)),
        out_shape=jax.ShapeDtypeStruct((NPAD, 128), jnp.float32),
    )(g1, gf, a2, mk, w2, b2, w3, b3)


def kernel(y, x, f_y, W1, b1, W2, b2, W3, b3):
    n = y.shape[0]
    m = x.shape[0]
    yord = jnp.argsort(y[:, 0]).astype(jnp.int32)
    xord = jnp.argsort(x[:, 0]).astype(jnp.int32)
    ysp = jnp.pad(jnp.pad(y[yord], ((0, 0), (0, 5))), ((0, NPAD - n), (0, 0)),
                  constant_values=100.0)
    xp = jnp.pad(jnp.pad(x[xord], ((0, 0), (0, 5))), ((0, NPAD - m), (0, 0)))
    yp = jnp.pad(jnp.pad(y, ((0, 0), (0, 5))), ((0, NPAD - n), (0, 0)),
                 constant_values=100.0)
    fyp = jnp.pad(f_y, ((0, NPAD - n), (0, 0)))
    lo = xp[::BM, 0] - RWIN
    st = jnp.searchsorted(ysp[:, 0], lo).astype(jnp.int32)
    start = jnp.clip(st, 0, NPAD - W)
    win = start[:, None] + jnp.arange(W, dtype=jnp.int32)[None, :]
    yrows = ysp[win]
    orow = jnp.pad(yord, (0, NPAD - n)).astype(jnp.float32)[win]
    ywin = jnp.concatenate(
        [yrows.transpose(0, 2, 1)[:, 0:3, :], orow[:, None, :],
         jnp.zeros((NPAD // BM, 4, W), jnp.float32)], axis=1)
    ywin = ywin.reshape(NPAD // BM * 8, W)

    A1 = _prep(yp, W1[:192], jnp.zeros((1, 128), jnp.float32))
    A2 = _prep(xp, W1[192:], b1[None, :])
    idxw, maskw = _search(xp, ywin)
    flat_idx = idxw[:, :K].T.reshape(-1)
    g1, gf = _make_gather()(A1, fyp, flat_idx)
    g1 = g1.reshape(K, NPAD, 128)
    gf = gf.reshape(K, NPAD, 128)
    out = _mlp(g1, gf, A2, maskw, W2, b2[None, :], W3, b3[None, :])
    inv = jnp.zeros((m,), jnp.int32).at[xord].set(
        jnp.arange(m, dtype=jnp.int32))
    return out[inv]


# speedup vs baseline: 4.4522x; 4.4522x over previous
import functools

import jax
import jax.numpy as jnp
from jax import lax
from jax.experimental import pallas as pl
from jax.experimental.pallas import tpu as pltpu
from jax.experimental.pallas import tpu_sc as plsc

RADIUS = 0.07
K = 12
NUM_FREQ = 32
MAX_POS = 10000.0
NPAD = 10240
BP = 1024
BM = 64
BQ = 256
NW = 32
EPW = NPAD * K // NW
CHUNK = 128


def _embed(c):
    kf = lax.broadcasted_iota(jnp.int32, (1, NUM_FREQ), 1).astype(jnp.float32)
    freqs = 1.0 / (MAX_POS ** (kf / NUM_FREQ))
    parts = []
    for d in range(3):
        ang = c[:, d:d + 1] * freqs
        parts.append(jnp.sin(ang))
        parts.append(jnp.cos(ang))
    return jnp.concatenate(parts, axis=1)


def _prep_body(c_ref, w_ref, b_ref, o_ref):
    emb = _embed(c_ref[...])
    o_ref[...] = (
        jnp.dot(emb, w_ref[...], preferred_element_type=jnp.float32) + b_ref[...]
    )


def _prep(coords, w, b):
    return pl.pallas_call(
        _prep_body,
        grid=(NPAD // BP,),
        in_specs=[
            pl.BlockSpec((BP, 8), lambda i: (i, 0)),
            pl.BlockSpec((192, 128), lambda i: (0, 0)),
            pl.BlockSpec((1, 128), lambda i: (0, 0)),
        ],
        out_specs=pl.BlockSpec((BP, 128), lambda i: (i, 0)),
        out_shape=jax.ShapeDtypeStruct((NPAD, 128), jnp.float32),
    )(coords, w, b)


RWIN = (RADIUS * RADIUS + 6.0 / 512.0) ** 0.5
W = 3584
NJ = W // 128
QSCALE = 131071.0 / (RADIUS * RADIUS)
IMAX = 2147483647


def _search_body(starts_ref, x_ref, y3_ref, o3_ref, idx_ref, mask_ref, key_ref):
    i = pl.program_id(0)
    s0 = starts_ref[i]
    xb = x_ref[...]
    xsq = (xb[:, 0:1] * xb[:, 0:1] + xb[:, 1:2] * xb[:, 1:2]) + xb[:, 2:3] * xb[:, 2:3]
    for c in range(NJ):
        yc = y3_ref[s0 + c]
        oc = o3_ref[s0 + c]
        ysq = (yc[0:1, :] * yc[0:1, :] + yc[1:2, :] * yc[1:2, :]) + yc[2:3, :] * yc[2:3, :]
        prod = jnp.dot(xb, yc, preferred_element_type=jnp.float32)
        d2 = (xsq + ysq) - 2.0 * prod
        d2 = jnp.maximum(d2, 0.0)
        q = (d2 * QSCALE).astype(jnp.int32)
        key_ref[:, c * 128:(c + 1) * 128] = jnp.where(
            d2 <= RADIUS * RADIUS, q * 16384 + oc, IMAX)
    keys = key_ref[...]
    idx_ref[...] = jnp.zeros((BM, 128), jnp.int32)
    mask_ref[...] = jnp.zeros((BM, 128), jnp.float32)
    for kk in range(K):
        cmin = jnp.min(keys, axis=1, keepdims=True)
        valid = cmin < IMAX
        oidx = jnp.bitwise_and(cmin, 16383)
        idx_ref[:, kk:kk + 1] = jnp.where(valid, oidx, 0)
        mask_ref[:, kk:kk + 1] = jnp.where(valid, 1.0, 0.0)
        keys = jnp.where(keys == cmin, IMAX, keys)


def _search(starts, xp, y3, o3):
    grid_spec = pltpu.PrefetchScalarGridSpec(
        num_scalar_prefetch=1,
        grid=(NPAD // BM,),
        in_specs=[
            pl.BlockSpec((BM, 8), lambda i, starts: (i, 0)),
            pl.BlockSpec((NPAD // 128, 8, 128), lambda i, starts: (0, 0, 0)),
            pl.BlockSpec((NPAD // 128, 1, 128), lambda i, starts: (0, 0, 0)),
        ],
        out_specs=[
            pl.BlockSpec((BM, 128), lambda i, starts: (i, 0)),
            pl.BlockSpec((BM, 128), lambda i, starts: (i, 0)),
        ],
        scratch_shapes=[pltpu.VMEM((BM, W), jnp.int32)],
    )
    return pl.pallas_call(
        _search_body,
        grid_spec=grid_spec,
        out_shape=[
            jax.ShapeDtypeStruct((NPAD, 128), jnp.int32),
            jax.ShapeDtypeStruct((NPAD, 128), jnp.float32),
        ],
    )(starts, xp, y3, o3)


NCH = EPW // CHUNK
NSLOT = 3


def _make_gather():
    mesh = plsc.VectorSubcoreMesh(core_axis_name="c", subcore_axis_name="s")

    @functools.partial(
        pl.kernel,
        mesh=mesh,
        out_type=(
            jax.ShapeDtypeStruct((NPAD * K, 128), jnp.float32),
            jax.ShapeDtypeStruct((NPAD * K, 128), jnp.float32),
        ),
        scratch_types=[
            pltpu.VMEM((EPW,), jnp.int32),
            pltpu.VMEM((NSLOT, CHUNK, 128), jnp.float32),
            pltpu.VMEM((NSLOT, CHUNK, 128), jnp.float32),
        ]
        + [pltpu.SemaphoreType.DMA] * (4 * NSLOT),
    )
    def gather(a1_hbm, fy_hbm, idx_hbm, g1_hbm, gf_hbm, idx_v, bufa, buff, *sems):
        ga = sems[0:NSLOT]
        gf = sems[NSLOT:2 * NSLOT]
        wsa = sems[2 * NSLOT:3 * NSLOT]
        wsf = sems[3 * NSLOT:4 * NSLOT]
        wid = lax.axis_index("s") * 2 + lax.axis_index("c")
        base = wid * EPW
        pltpu.sync_copy(idx_hbm.at[pl.ds(base, EPW)], idx_v)

        hA = [None] * NSLOT
        hF = [None] * NSLOT
        wA = [None] * NSLOT
        wF = [None] * NSLOT

        def g_start(c, s):
            isl = idx_v.at[pl.ds(c * CHUNK, CHUNK)]
            hA[s] = pltpu.async_copy(a1_hbm.at[isl], bufa.at[s], ga[s])
            hF[s] = pltpu.async_copy(fy_hbm.at[isl], buff.at[s], gf[s])

        for c in range(min(NSLOT, NCH)):
            g_start(c, c)
        for c in range(NCH):
            s = c % NSLOT
            hA[s].wait()
            hF[s].wait()
            wA[s] = pltpu.async_copy(
                bufa.at[s], g1_hbm.at[pl.ds(base + c * CHUNK, CHUNK)], wsa[s])
            wF[s] = pltpu.async_copy(
                buff.at[s], gf_hbm.at[pl.ds(base + c * CHUNK, CHUNK)], wsf[s])
            nc = c + NSLOT
            if nc < NCH:
                wA[s].wait()
                wF[s].wait()
                g_start(nc, s)
        for c in range(max(NCH - NSLOT, 0), NCH):
            s = c % NSLOT
            wA[s].wait()
            wF[s].wait()

    return gather


def _mlp_body(g1_ref, gf_ref, a2_ref, mk_ref, w2_ref, b2_ref, w3_ref, b3_ref, o_ref):
    a2 = a2_ref[...]
    w2 = w2_ref[...]
    b2 = b2_ref[...]
    w3 = w3_ref[...]
    b3 = b3_ref[...]
    mk = mk_ref[...]
    acc = jnp.zeros((BQ, 128), jnp.float32)
    for kk in range(K):
        h = jax.nn.gelu(g1_ref[kk] + a2)
        h = jax.nn.gelu(jnp.dot(h, w2, preferred_element_type=jnp.float32) + b2)
        kv = (jnp.dot(h, w3, preferred_element_type=jnp.float32) + b3) * gf_ref[kk]
        acc = acc + kv * mk[:, kk:kk + 1]
    o_ref[...] = acc


def _mlp(g1, gf, a2, mk, w2, b2, w3, b3):
    return pl.pallas_call(
        _mlp_body,
        grid=(NPAD // BQ,),
        in_specs=[
            pl.BlockSpec((K, BQ, 128), lambda i: (0, i, 0)),
            pl.BlockSpec((K, BQ, 128), lambda i: (0, i, 0)),
            pl.BlockSpec((BQ, 128), lambda i: (i, 0)),
            pl.BlockSpec((BQ, 128), lambda i: (i, 0)),
            pl.BlockSpec((128, 256), lambda i: (0, 0)),
            pl.BlockSpec((1, 256), lambda i: (0, 0)),
            pl.BlockSpec((256, 128), lambda i: (0, 0)),
            pl.BlockSpec((1, 128), lambda i: (0, 0)),
        ],
        out_specs=pl.BlockSpec((BQ, 128), lambda i: (i, 0)),
        out_shape=jax.ShapeDtypeStruct((NPAD, 128), jnp.float32),
    )(g1, gf, a2, mk, w2, b2, w3, b3)


def kernel(y, x, f_y, W1, b1, W2, b2, W3, b3):
    n = y.shape[0]
    m = x.shape[0]
    yord = jnp.argsort(y[:, 0]).astype(jnp.int32)
    xord = jnp.argsort(x[:, 0]).astype(jnp.int32)
    ysp = jnp.pad(jnp.pad(y[yord], ((0, 0), (0, 5))), ((0, NPAD - n), (0, 0)),
                  constant_values=100.0)
    xp = jnp.pad(jnp.pad(x[xord], ((0, 0), (0, 5))), ((0, NPAD - m), (0, 0)))
    yp = jnp.pad(jnp.pad(y, ((0, 0), (0, 5))), ((0, NPAD - n), (0, 0)),
                 constant_values=100.0)
    fyp = jnp.pad(f_y, ((0, NPAD - n), (0, 0)))
    lo = xp[::BM, 0] - RWIN
    st = jnp.searchsorted(ysp[:, 0], lo).astype(jnp.int32)
    starts = jnp.clip(st // 128, 0, (NPAD - W) // 128)
    y3 = ysp.T.reshape(8, NPAD // 128, 128).transpose(1, 0, 2)
    o3 = jnp.pad(yord, (0, NPAD - n)).reshape(NPAD // 128, 1, 128)

    A1 = _prep(yp, W1[:192], jnp.zeros((1, 128), jnp.float32))
    A2 = _prep(xp, W1[192:], b1[None, :])
    idxw, maskw = _search(starts, xp, y3, o3)
    flat_idx = idxw[:, :K].T.reshape(-1)
    g1, gf = _make_gather()(A1, fyp, flat_idx)
    g1 = g1.reshape(K, NPAD, 128)
    gf = gf.reshape(K, NPAD, 128)
    out = _mlp(g1, gf, A2, maskw, W2, b2[None, :], W3, b3[None, :])
    inv = jnp.zeros((m,), jnp.int32).at[xord].set(
        jnp.arange(m, dtype=jnp.int32))
    return out[inv]
